# Initial kernel scaffold; baseline (speedup 1.0000x reference)
import jax
import jax.numpy as jnp
from jax.experimental import pallas as pl


def kernel(x, proj_w, proj_b, merge_w, merge_b, alpha, beta, norm0_w, norm0_b, lin0_w, lin0_b, lin1_w, lin1_b, norm1_w, norm1_b):
    raise NotImplementedError("write your pallas kernel here")


# speedup vs baseline: 3.0394x; 3.0394x over previous
import jax
import jax.numpy as jnp
from jax.experimental import pallas as pl
from jax.experimental.pallas import tpu as pltpu

_N, _C, _H, _W = 2, 384, 112, 112
_HID = 384
_FC = 8
_CS = 7
_FS = 2
_HQ = _H // _FS
_HW = _HQ * _HQ
_CG = 2 * _HID // _FC
_HALF = _CG // 2
_S = _CS * _CS
_KH = _HQ // _CS
_CNT = _C * _H * _W
_KT = 1
_TW = _HW // _KT


def _cluster_body(alpha_ref, beta_ref, x_ref, pw_ref, pb_ref, mw_ref, mb_ref,
                  merged_ref, s1_ref, s2_ref):
    fci = pl.program_id(3)
    xq = x_ref[0, 0, 0]
    wg = pw_ref[...]
    bg = pb_ref[...]
    hi_p = jax.lax.Precision.HIGHEST
    bf = jnp.bfloat16
    y = jnp.dot(wg.astype(bf), xq.astype(bf),
                preferred_element_type=jnp.float32) + bg

    li = jax.lax.broadcasted_iota(jnp.int32, (_HW, _S), 0)
    ji = jax.lax.broadcasted_iota(jnp.int32, (_HW, _S), 1)
    hi = li // _HQ
    wi = li - hi * _HQ
    blk = (hi // _KH) * _CS + (wi // _KH)
    q = jnp.where(blk == ji, 1.0 / (_KH * _KH), 0.0)
    cen = jnp.dot(y, q, precision=hi_p,
                  preferred_element_type=jnp.float32)

    xp = y[:_HALF, :]
    xv = y[_HALF:, :]
    cp = cen[:_HALF, :]
    cv = cen[_HALF:, :]

    dx = jnp.maximum(jnp.sqrt(jnp.sum(xp * xp, axis=0, keepdims=True)), 1e-12)
    nx = xp / dx
    dc = jnp.maximum(jnp.sqrt(jnp.sum(cp * cp, axis=0, keepdims=True)), 1e-12)
    nc = cp / dc

    simT = jax.lax.dot_general(nc.astype(bf), nx.astype(bf),
                               (((0,), (0,)), ((), ())),
                               preferred_element_type=jnp.float32)
    simT = jax.nn.sigmoid(alpha_ref[0, 0] * simT + beta_ref[0, 0])
    vals = jnp.max(simT, axis=0, keepdims=True)
    ismax = (simT >= vals).astype(jnp.float32)
    ri = jax.lax.broadcasted_iota(jnp.int32, (_S, _S), 0)
    ci = jax.lax.broadcasted_iota(jnp.int32, (_S, _S), 1)
    ltri = jnp.where(ri >= ci, 1.0, 0.0)
    cum = jnp.dot(ltri, ismax, precision=hi_p,
                  preferred_element_type=jnp.float32)
    w1 = jnp.where(cum == 1.0, ismax, 0.0) * vals

    delta = jax.lax.dot_general(xv, w1, (((1,), (1,)), ((), ())), precision=hi_p,
                                preferred_element_type=jnp.float32)
    ones_row = jnp.ones((1, _HW), jnp.float32)
    wsum = jax.lax.dot_general(ones_row, w1, (((1,), (1,)), ((), ())),
                               precision=hi_p,
                               preferred_element_type=jnp.float32)
    ncf = (cv + delta) / (1.0 + wsum)
    nxq = jnp.dot(ncf, w1, precision=hi_p,
                  preferred_element_type=jnp.float32)

    mwg = mw_ref[0]
    contrib = jnp.dot(mwg.astype(bf), nxq.astype(bf),
                      preferred_element_type=jnp.float32)

    @pl.when(fci == 0)
    def _():
        merged_ref[0, 0, 0] = contrib + mb_ref[...]

    @pl.when(fci > 0)
    def _():
        merged_ref[0, 0, 0] = merged_ref[0, 0, 0] + contrib

    @pl.when(fci == _FC - 1)
    def _():
        mm = merged_ref[0, 0, 0]
        s1_ref[0, 0, 0] = jnp.sum(mm)
        s2_ref[0, 0, 0] = jnp.sum(mm * mm)


def _flat3(n, h, w, k=None):
    if k is None:
        return n * 4 + h * 2 + w
    return (n * 4 + h * 2 + w) * _KT + k


def _mlp_body(s1_ref, s2_ref, x_ref, m_ref, n0w_ref, n0b_ref, w0_ref, b0_ref,
              w1_ref, b1_ref, t_ref, p1_ref, p2_ref):
    ni = pl.program_id(0)
    tot = 0.0
    totsq = 0.0
    for i in range(2):
        for j in range(2):
            tot = tot + s1_ref[ni * 4 + i * 2 + j, 0, 0]
            totsq = totsq + s2_ref[ni * 4 + i * 2 + j, 0, 0]
    mu = tot / _CNT
    var = totsq / _CNT - mu * mu
    inv = jax.lax.rsqrt(var + 1e-5)

    bf = jnp.bfloat16
    g = (m_ref[0, 0, 0] - mu) * inv * n0w_ref[...] + n0b_ref[...]
    xt = x_ref[0, 0, 0]
    w0 = w0_ref[...].astype(bf)
    h1 = (jnp.dot(w0[:, :_C], xt.astype(bf), preferred_element_type=jnp.float32)
          + jnp.dot(w0[:, _C:], g.astype(bf), preferred_element_type=jnp.float32)
          + b0_ref[...])
    h1 = 0.5 * h1 * (1.0 + jax.lax.erf(h1 * 0.7071067811865476))
    t = (jnp.dot(w1_ref[...].astype(bf), h1.astype(bf),
                 preferred_element_type=jnp.float32) + b1_ref[...])
    t_ref[0, 0, 0] = t
    p1_ref[0, 0, 0] = jnp.sum(t)
    p2_ref[0, 0, 0] = jnp.sum(t * t)


def _final_body(p1_ref, p2_ref, n1w_ref, n1b_ref, t_ref, x_ref, o_ref):
    ni = pl.program_id(0)
    tot = 0.0
    totsq = 0.0
    for i in range(2):
        for j in range(2):
            for k in range(_KT):
                idx = (ni * 4 + i * 2 + j) * _KT + k
                tot = tot + p1_ref[idx, 0, 0]
                totsq = totsq + p2_ref[idx, 0, 0]
    mu = tot / _CNT
    var = totsq / _CNT - mu * mu
    inv = jax.lax.rsqrt(var + 1e-5)
    o_ref[0, 0, 0] = ((t_ref[0, 0, 0] - mu) * inv * n1w_ref[...]
                      + n1b_ref[...] + x_ref[0, 0, 0])


def kernel(x, proj_w, proj_b, merge_w, merge_b, alpha, beta, norm0_w, norm0_b,
           lin0_w, lin0_b, lin1_w, lin1_b, norm1_w, norm1_b):
    f32 = jnp.float32
    x_q = (x.reshape(_N, _C, _FS, _HQ, _FS, _HQ)
           .transpose(0, 2, 4, 1, 3, 5).reshape(_N, _FS, _FS, _C, _HW))
    pb2 = proj_b.reshape(2 * _HID, 1)
    mw3 = merge_w.reshape(_C, _FC, _HALF).transpose(1, 0, 2)
    mb2 = merge_b.reshape(_C, 1)
    a2 = alpha.reshape(1, 1)
    b2 = beta.reshape(1, 1)

    smem = pl.BlockSpec(memory_space=pltpu.SMEM)
    merged, s1, s2 = pl.pallas_call(
        _cluster_body,
        grid=(_N, _FS, _FS, _FC),
        in_specs=[
            smem, smem,
            pl.BlockSpec((1, 1, 1, _C, _HW), lambda n, h, w, f: (n, h, w, 0, 0)),
            pl.BlockSpec((_CG, _C), lambda n, h, w, f: (f, 0)),
            pl.BlockSpec((_CG, 1), lambda n, h, w, f: (f, 0)),
            pl.BlockSpec((1, _C, _HALF), lambda n, h, w, f: (f, 0, 0)),
            pl.BlockSpec((_C, 1), lambda n, h, w, f: (0, 0)),
        ],
        out_specs=[
            pl.BlockSpec((1, 1, 1, _C, _HW), lambda n, h, w, f: (n, h, w, 0, 0)),
            pl.BlockSpec((1, 1, 1), lambda n, h, w, f: (_flat3(n, h, w), 0, 0),
                         memory_space=pltpu.SMEM),
            pl.BlockSpec((1, 1, 1), lambda n, h, w, f: (_flat3(n, h, w), 0, 0),
                         memory_space=pltpu.SMEM),
        ],
        out_shape=[
            jax.ShapeDtypeStruct((_N, _FS, _FS, _C, _HW), f32),
            jax.ShapeDtypeStruct((_N * _FS * _FS, 1, 1), f32),
            jax.ShapeDtypeStruct((_N * _FS * _FS, 1, 1), f32),
        ],
        compiler_params=pltpu.CompilerParams(
            dimension_semantics=("parallel", "parallel", "parallel", "arbitrary")),
    )(a2, b2, x_q, proj_w, pb2, mw3, mb2)

    t_all, p1, p2 = pl.pallas_call(
        _mlp_body,
        grid=(_N, _FS, _FS, _KT),
        in_specs=[
            smem, smem,
            pl.BlockSpec((1, 1, 1, _C, _TW), lambda n, h, w, k: (n, h, w, 0, k)),
            pl.BlockSpec((1, 1, 1, _C, _TW), lambda n, h, w, k: (n, h, w, 0, k)),
            pl.BlockSpec((_C, 1), lambda n, h, w, k: (0, 0)),
            pl.BlockSpec((_C, 1), lambda n, h, w, k: (0, 0)),
            pl.BlockSpec((2 * _C, 2 * _C), lambda n, h, w, k: (0, 0)),
            pl.BlockSpec((2 * _C, 1), lambda n, h, w, k: (0, 0)),
            pl.BlockSpec((_C, 2 * _C), lambda n, h, w, k: (0, 0)),
            pl.BlockSpec((_C, 1), lambda n, h, w, k: (0, 0)),
        ],
        out_specs=[
            pl.BlockSpec((1, 1, 1, _C, _TW), lambda n, h, w, k: (n, h, w, 0, k)),
            pl.BlockSpec((1, 1, 1), lambda n, h, w, k: (_flat3(n, h, w, k), 0, 0),
                         memory_space=pltpu.SMEM),
            pl.BlockSpec((1, 1, 1), lambda n, h, w, k: (_flat3(n, h, w, k), 0, 0),
                         memory_space=pltpu.SMEM),
        ],
        out_shape=[
            jax.ShapeDtypeStruct((_N, _FS, _FS, _C, _HW), f32),
            jax.ShapeDtypeStruct((_N * _FS * _FS * _KT, 1, 1), f32),
            jax.ShapeDtypeStruct((_N * _FS * _FS * _KT, 1, 1), f32),
        ],
        compiler_params=pltpu.CompilerParams(
            dimension_semantics=("parallel", "parallel", "parallel", "parallel")),
    )(s1, s2, x_q, merged, norm0_w.reshape(_C, 1), norm0_b.reshape(_C, 1),
      lin0_w, lin0_b.reshape(2 * _C, 1), lin1_w, lin1_b.reshape(_C, 1))

    out_q = pl.pallas_call(
        _final_body,
        grid=(_N, _FS, _FS, _KT),
        in_specs=[
            smem, smem,
            pl.BlockSpec((_C, 1), lambda n, h, w, k: (0, 0)),
            pl.BlockSpec((_C, 1), lambda n, h, w, k: (0, 0)),
            pl.BlockSpec((1, 1, 1, _C, _TW), lambda n, h, w, k: (n, h, w, 0, k)),
            pl.BlockSpec((1, 1, 1, _C, _TW), lambda n, h, w, k: (n, h, w, 0, k)),
        ],
        out_specs=pl.BlockSpec((1, 1, 1, _C, _TW),
                               lambda n, h, w, k: (n, h, w, 0, k)),
        out_shape=jax.ShapeDtypeStruct((_N, _FS, _FS, _C, _HW), f32),
        compiler_params=pltpu.CompilerParams(
            dimension_semantics=("parallel", "parallel", "parallel", "parallel")),
    )(p1, p2, norm1_w.reshape(_C, 1), norm1_b.reshape(_C, 1), t_all, x_q)

    return (out_q.reshape(_N, _FS, _FS, _C, _HQ, _HQ)
            .transpose(0, 3, 1, 4, 2, 5).reshape(_N, _C, _H, _W))


# speedup vs baseline: 3.3433x; 1.1000x over previous
import jax
import jax.numpy as jnp
from jax.experimental import pallas as pl
from jax.experimental.pallas import tpu as pltpu

_N, _C, _H, _W = 2, 384, 112, 112
_HID = 384
_FC = 8
_CS = 7
_FS = 2
_HQ = _H // _FS
_HW = _HQ * _HQ
_CG = 2 * _HID // _FC
_HALF = _CG // 2
_S = _CS * _CS
_KH = _HQ // _CS
_CNT = _C * _H * _W
_KT = 1
_TW = _HW // _KT


def _dot3(a, b, dims):
    f32 = jnp.float32
    bf = jnp.bfloat16
    ah = a.astype(bf)
    al = (a - ah.astype(f32)).astype(bf)
    bh = b.astype(bf)
    bl = (b - bh.astype(f32)).astype(bf)
    return (jax.lax.dot_general(ah, bh, dims, preferred_element_type=f32)
            + jax.lax.dot_general(ah, bl, dims, preferred_element_type=f32)
            + jax.lax.dot_general(al, bh, dims, preferred_element_type=f32))


def _cluster_body(alpha_ref, beta_ref, x_ref, pw_ref, pb_ref, mw_ref, mb_ref,
                  merged_ref, s1_ref, s2_ref, xbf_ref):
    fci = pl.program_id(3)
    wg = pw_ref[...]
    bg = pb_ref[...]
    hi_p = jax.lax.Precision.HIGHEST
    bf = jnp.bfloat16

    @pl.when(fci == 0)
    def _():
        xbf_ref[...] = x_ref[0, 0, 0].astype(bf)

    y = jnp.dot(wg.astype(bf), xbf_ref[...],
                preferred_element_type=jnp.float32) + bg

    li = jax.lax.broadcasted_iota(jnp.int32, (_HW, _S), 0)
    ji = jax.lax.broadcasted_iota(jnp.int32, (_HW, _S), 1)
    hi = li // _HQ
    wi = li - hi * _HQ
    blk = (hi // _KH) * _CS + (wi // _KH)
    q = jnp.where(blk == ji, 1.0 / (_KH * _KH), 0.0)
    cen = jnp.dot(y, q, precision=hi_p,
                  preferred_element_type=jnp.float32)

    xp = y[:_HALF, :]
    xv = y[_HALF:, :]
    cp = cen[:_HALF, :]
    cv = cen[_HALF:, :]

    dx = jnp.maximum(jnp.sqrt(jnp.sum(xp * xp, axis=0, keepdims=True)), 1e-12)
    nx = xp / dx
    dc = jnp.maximum(jnp.sqrt(jnp.sum(cp * cp, axis=0, keepdims=True)), 1e-12)
    nc = cp / dc

    simT = jax.lax.dot_general(nc.astype(bf), nx.astype(bf),
                               (((0,), (0,)), ((), ())),
                               preferred_element_type=jnp.float32)
    simT = jax.nn.sigmoid(alpha_ref[0, 0] * simT + beta_ref[0, 0])
    vals = jnp.max(simT, axis=0, keepdims=True)
    ismax = (simT >= vals).astype(jnp.float32)
    ri = jax.lax.broadcasted_iota(jnp.int32, (_S, _S), 0)
    ci = jax.lax.broadcasted_iota(jnp.int32, (_S, _S), 1)
    ltri = jnp.where(ri >= ci, 1.0, 0.0).astype(bf)
    cum = jnp.dot(ltri, ismax.astype(bf),
                  preferred_element_type=jnp.float32)
    w1 = jnp.where(cum == 1.0, ismax, 0.0) * vals

    xve = jnp.concatenate([xv, jnp.ones((1, _HW), jnp.float32)], axis=0)
    delta = _dot3(xve, w1, (((1,), (1,)), ((), ())))
    ncf = (cv + delta[:_HALF, :]) / (1.0 + delta[_HALF:, :])
    nxq = _dot3(ncf, w1, (((1,), (0,)), ((), ())))

    mwg = mw_ref[0]
    contrib = jnp.dot(mwg.astype(bf), nxq.astype(bf),
                      preferred_element_type=jnp.float32)

    @pl.when(fci == 0)
    def _():
        merged_ref[0, 0, 0] = contrib + mb_ref[...]

    @pl.when(fci > 0)
    def _():
        merged_ref[0, 0, 0] = merged_ref[0, 0, 0] + contrib

    @pl.when(fci == _FC - 1)
    def _():
        mm = merged_ref[0, 0, 0]
        s1_ref[0, 0, 0] = jnp.sum(mm)
        s2_ref[0, 0, 0] = jnp.sum(mm * mm)


def _flat3(n, h, w, k=None):
    if k is None:
        return n * 4 + h * 2 + w
    return (n * 4 + h * 2 + w) * _KT + k


def _mlp_body(s1_ref, s2_ref, x_ref, m_ref, n0w_ref, n0b_ref, w0_ref, b0_ref,
              w1_ref, b1_ref, t_ref, p1_ref, p2_ref):
    ni = pl.program_id(0)
    tot = 0.0
    totsq = 0.0
    for i in range(2):
        for j in range(2):
            tot = tot + s1_ref[ni * 4 + i * 2 + j, 0, 0]
            totsq = totsq + s2_ref[ni * 4 + i * 2 + j, 0, 0]
    mu = tot / _CNT
    var = totsq / _CNT - mu * mu
    inv = jax.lax.rsqrt(var + 1e-5)

    bf = jnp.bfloat16
    g = (m_ref[0, 0, 0] - mu) * inv * n0w_ref[...] + n0b_ref[...]
    xt = x_ref[0, 0, 0]
    w0 = w0_ref[...].astype(bf)
    h1 = (jnp.dot(w0[:, :_C], xt.astype(bf), preferred_element_type=jnp.float32)
          + jnp.dot(w0[:, _C:], g.astype(bf), preferred_element_type=jnp.float32)
          + b0_ref[...])
    h1 = 0.5 * h1 * (1.0 + jax.lax.erf(h1 * 0.7071067811865476))
    t = (jnp.dot(w1_ref[...].astype(bf), h1.astype(bf),
                 preferred_element_type=jnp.float32) + b1_ref[...])
    t_ref[0, 0, 0] = t
    p1_ref[0, 0, 0] = jnp.sum(t)
    p2_ref[0, 0, 0] = jnp.sum(t * t)


def _final_body(p1_ref, p2_ref, n1w_ref, n1b_ref, t_ref, x_ref, o_ref):
    ni = pl.program_id(0)
    tot = 0.0
    totsq = 0.0
    for i in range(2):
        for j in range(2):
            for k in range(_KT):
                idx = (ni * 4 + i * 2 + j) * _KT + k
                tot = tot + p1_ref[idx, 0, 0]
                totsq = totsq + p2_ref[idx, 0, 0]
    mu = tot / _CNT
    var = totsq / _CNT - mu * mu
    inv = jax.lax.rsqrt(var + 1e-5)
    o_ref[0, 0, 0] = ((t_ref[0, 0, 0] - mu) * inv * n1w_ref[...]
                      + n1b_ref[...] + x_ref[0, 0, 0])


def kernel(x, proj_w, proj_b, merge_w, merge_b, alpha, beta, norm0_w, norm0_b,
           lin0_w, lin0_b, lin1_w, lin1_b, norm1_w, norm1_b):
    f32 = jnp.float32
    x_q = (x.reshape(_N, _C, _FS, _HQ, _FS, _HQ)
           .transpose(0, 2, 4, 1, 3, 5).reshape(_N, _FS, _FS, _C, _HW))
    pb2 = proj_b.reshape(2 * _HID, 1)
    mw3 = merge_w.reshape(_C, _FC, _HALF).transpose(1, 0, 2)
    mb2 = merge_b.reshape(_C, 1)
    a2 = alpha.reshape(1, 1)
    b2 = beta.reshape(1, 1)

    smem = pl.BlockSpec(memory_space=pltpu.SMEM)
    merged, s1, s2 = pl.pallas_call(
        _cluster_body,
        grid=(_N, _FS, _FS, _FC),
        in_specs=[
            smem, smem,
            pl.BlockSpec((1, 1, 1, _C, _HW), lambda n, h, w, f: (n, h, w, 0, 0)),
            pl.BlockSpec((_CG, _C), lambda n, h, w, f: (f, 0)),
            pl.BlockSpec((_CG, 1), lambda n, h, w, f: (f, 0)),
            pl.BlockSpec((1, _C, _HALF), lambda n, h, w, f: (f, 0, 0)),
            pl.BlockSpec((_C, 1), lambda n, h, w, f: (0, 0)),
        ],
        out_specs=[
            pl.BlockSpec((1, 1, 1, _C, _HW), lambda n, h, w, f: (n, h, w, 0, 0)),
            pl.BlockSpec((1, 1, 1), lambda n, h, w, f: (_flat3(n, h, w), 0, 0),
                         memory_space=pltpu.SMEM),
            pl.BlockSpec((1, 1, 1), lambda n, h, w, f: (_flat3(n, h, w), 0, 0),
                         memory_space=pltpu.SMEM),
        ],
        out_shape=[
            jax.ShapeDtypeStruct((_N, _FS, _FS, _C, _HW), f32),
            jax.ShapeDtypeStruct((_N * _FS * _FS, 1, 1), f32),
            jax.ShapeDtypeStruct((_N * _FS * _FS, 1, 1), f32),
        ],
        scratch_shapes=[pltpu.VMEM((_C, _HW), jnp.bfloat16)],
        compiler_params=pltpu.CompilerParams(
            dimension_semantics=("parallel", "parallel", "parallel", "arbitrary")),
    )(a2, b2, x_q, proj_w, pb2, mw3, mb2)

    t_all, p1, p2 = pl.pallas_call(
        _mlp_body,
        grid=(_N, _FS, _FS, _KT),
        in_specs=[
            smem, smem,
            pl.BlockSpec((1, 1, 1, _C, _TW), lambda n, h, w, k: (n, h, w, 0, k)),
            pl.BlockSpec((1, 1, 1, _C, _TW), lambda n, h, w, k: (n, h, w, 0, k)),
            pl.BlockSpec((_C, 1), lambda n, h, w, k: (0, 0)),
            pl.BlockSpec((_C, 1), lambda n, h, w, k: (0, 0)),
            pl.BlockSpec((2 * _C, 2 * _C), lambda n, h, w, k: (0, 0)),
            pl.BlockSpec((2 * _C, 1), lambda n, h, w, k: (0, 0)),
            pl.BlockSpec((_C, 2 * _C), lambda n, h, w, k: (0, 0)),
            pl.BlockSpec((_C, 1), lambda n, h, w, k: (0, 0)),
        ],
        out_specs=[
            pl.BlockSpec((1, 1, 1, _C, _TW), lambda n, h, w, k: (n, h, w, 0, k)),
            pl.BlockSpec((1, 1, 1), lambda n, h, w, k: (_flat3(n, h, w, k), 0, 0),
                         memory_space=pltpu.SMEM),
            pl.BlockSpec((1, 1, 1), lambda n, h, w, k: (_flat3(n, h, w, k), 0, 0),
                         memory_space=pltpu.SMEM),
        ],
        out_shape=[
            jax.ShapeDtypeStruct((_N, _FS, _FS, _C, _HW), f32),
            jax.ShapeDtypeStruct((_N * _FS * _FS * _KT, 1, 1), f32),
            jax.ShapeDtypeStruct((_N * _FS * _FS * _KT, 1, 1), f32),
        ],
        compiler_params=pltpu.CompilerParams(
            dimension_semantics=("parallel", "parallel", "parallel", "parallel")),
    )(s1, s2, x_q, merged, norm0_w.reshape(_C, 1), norm0_b.reshape(_C, 1),
      lin0_w, lin0_b.reshape(2 * _C, 1), lin1_w, lin1_b.reshape(_C, 1))

    out_q = pl.pallas_call(
        _final_body,
        grid=(_N, _FS, _FS, _KT),
        in_specs=[
            smem, smem,
            pl.BlockSpec((_C, 1), lambda n, h, w, k: (0, 0)),
            pl.BlockSpec((_C, 1), lambda n, h, w, k: (0, 0)),
            pl.BlockSpec((1, 1, 1, _C, _TW), lambda n, h, w, k: (n, h, w, 0, k)),
            pl.BlockSpec((1, 1, 1, _C, _TW), lambda n, h, w, k: (n, h, w, 0, k)),
        ],
        out_specs=pl.BlockSpec((1, 1, 1, _C, _TW),
                               lambda n, h, w, k: (n, h, w, 0, k)),
        out_shape=jax.ShapeDtypeStruct((_N, _FS, _FS, _C, _HW), f32),
        compiler_params=pltpu.CompilerParams(
            dimension_semantics=("parallel", "parallel", "parallel", "parallel")),
    )(p1, p2, norm1_w.reshape(_C, 1), norm1_b.reshape(_C, 1), t_all, x_q)

    return (out_q.reshape(_N, _FS, _FS, _C, _HQ, _HQ)
            .transpose(0, 3, 1, 4, 2, 5).reshape(_N, _C, _H, _W))


# speedup vs baseline: 4.5318x; 1.3555x over previous
import jax
import jax.numpy as jnp
from jax.experimental import pallas as pl
from jax.experimental.pallas import tpu as pltpu

_N, _C, _H, _W = 2, 384, 112, 112
_HID = 384
_FC = 8
_CS = 7
_FS = 2
_HQ = _H // _FS
_HW = _HQ * _HQ
_CG = 2 * _HID // _FC
_HALF = _CG // 2
_S = _CS * _CS
_KH = _HQ // _CS
_CNT = _C * _H * _W
_KT = 1
_TW = _HW // _KT


def _dot3(a, b, dims):
    f32 = jnp.float32
    bf = jnp.bfloat16
    ah = a.astype(bf)
    al = (a - ah.astype(f32)).astype(bf)
    bh = b.astype(bf)
    bl = (b - bh.astype(f32)).astype(bf)
    return (jax.lax.dot_general(ah, bh, dims, preferred_element_type=f32)
            + jax.lax.dot_general(ah, bl, dims, preferred_element_type=f32)
            + jax.lax.dot_general(al, bh, dims, preferred_element_type=f32))


def _cluster_body(alpha_ref, beta_ref, x_ref, pw_ref, pb_ref, mw_ref, mb_ref,
                  q_ref, merged_ref, s1_ref, s2_ref, xbf_ref):
    fci = pl.program_id(3)
    wg = pw_ref[...]
    bg = pb_ref[...]
    hi_p = jax.lax.Precision.HIGHEST
    bf = jnp.bfloat16

    @pl.when(fci == 0)
    def _():
        xbf_ref[...] = x_ref[0, 0, 0].astype(bf)

    y = jnp.dot(wg.astype(bf), xbf_ref[...],
                preferred_element_type=jnp.float32) + bg

    q = q_ref[...]
    yh = y.astype(bf)
    r1 = y - yh.astype(jnp.float32)
    ym = r1.astype(bf)
    yl = (r1 - ym.astype(jnp.float32)).astype(bf)
    cen = (jnp.dot(yh, q, preferred_element_type=jnp.float32)
           + jnp.dot(ym, q, preferred_element_type=jnp.float32)
           + jnp.dot(yl, q, preferred_element_type=jnp.float32))

    xp = y[:_HALF, :]
    xv = y[_HALF:, :]
    cp = cen[:_HALF, :]
    cv = cen[_HALF:, :]

    dx = jnp.maximum(jnp.sqrt(jnp.sum(xp * xp, axis=0, keepdims=True)), 1e-12)
    nx = xp / dx
    dc = jnp.maximum(jnp.sqrt(jnp.sum(cp * cp, axis=0, keepdims=True)), 1e-12)
    nc = cp / dc

    simT = jax.lax.dot_general(nc.astype(bf), nx.astype(bf),
                               (((0,), (0,)), ((), ())),
                               preferred_element_type=jnp.float32)
    simT = jax.nn.sigmoid(alpha_ref[0, 0] * simT + beta_ref[0, 0])
    vals = jnp.max(simT, axis=0, keepdims=True)
    ismax = (simT >= vals).astype(jnp.float32)
    ri = jax.lax.broadcasted_iota(jnp.int32, (_S, _S), 0)
    ci = jax.lax.broadcasted_iota(jnp.int32, (_S, _S), 1)
    ltri = jnp.where(ri >= ci, 1.0, 0.0).astype(bf)
    cum = jnp.dot(ltri, ismax.astype(bf),
                  preferred_element_type=jnp.float32)
    w1 = jnp.where(cum == 1.0, ismax, 0.0) * vals

    xve = jnp.concatenate([xv, jnp.ones((1, _HW), jnp.float32)], axis=0)
    delta = _dot3(xve, w1, (((1,), (1,)), ((), ())))
    ncf = (cv + delta[:_HALF, :]) / (1.0 + delta[_HALF:, :])
    nxq = _dot3(ncf, w1, (((1,), (0,)), ((), ())))

    mwg = mw_ref[0]
    contrib = jnp.dot(mwg.astype(bf), nxq.astype(bf),
                      preferred_element_type=jnp.float32)

    @pl.when(fci == 0)
    def _():
        merged_ref[0, 0, 0] = contrib + mb_ref[...]

    @pl.when(fci > 0)
    def _():
        merged_ref[0, 0, 0] = merged_ref[0, 0, 0] + contrib

    @pl.when(fci == _FC - 1)
    def _():
        mm = merged_ref[0, 0, 0]
        s1_ref[0, 0, 0] = jnp.sum(mm)
        s2_ref[0, 0, 0] = jnp.sum(mm * mm)


def _flat3(n, h, w, k=None):
    if k is None:
        return n * 4 + h * 2 + w
    return (n * 4 + h * 2 + w) * _KT + k


def _mlp_body(s1_ref, s2_ref, x_ref, m_ref, n0w_ref, n0b_ref, w0_ref, b0_ref,
              w1_ref, b1_ref, t_ref, p1_ref, p2_ref):
    ni = pl.program_id(0)
    tot = 0.0
    totsq = 0.0
    for i in range(2):
        for j in range(2):
            tot = tot + s1_ref[ni * 4 + i * 2 + j, 0, 0]
            totsq = totsq + s2_ref[ni * 4 + i * 2 + j, 0, 0]
    mu = tot / _CNT
    var = totsq / _CNT - mu * mu
    inv = jax.lax.rsqrt(var + 1e-5)

    bf = jnp.bfloat16
    g = (m_ref[0, 0, 0] - mu) * inv * n0w_ref[...] + n0b_ref[...]
    xt = x_ref[0, 0, 0]
    w0 = w0_ref[...].astype(bf)
    h1 = (jnp.dot(w0[:, :_C], xt.astype(bf), preferred_element_type=jnp.float32)
          + jnp.dot(w0[:, _C:], g.astype(bf), preferred_element_type=jnp.float32)
          + b0_ref[...])
    h1 = 0.5 * h1 * (1.0 + jax.lax.erf(h1 * 0.7071067811865476))
    t = (jnp.dot(w1_ref[...].astype(bf), h1.astype(bf),
                 preferred_element_type=jnp.float32) + b1_ref[...])
    t_ref[0, 0, 0] = t
    p1_ref[0, 0, 0] = jnp.sum(t)
    p2_ref[0, 0, 0] = jnp.sum(t * t)


def _final_body(p1_ref, p2_ref, n1w_ref, n1b_ref, t_ref, x_ref, o_ref):
    ni = pl.program_id(0)
    tot = 0.0
    totsq = 0.0
    for i in range(2):
        for j in range(2):
            for k in range(_KT):
                idx = (ni * 4 + i * 2 + j) * _KT + k
                tot = tot + p1_ref[idx, 0, 0]
                totsq = totsq + p2_ref[idx, 0, 0]
    mu = tot / _CNT
    var = totsq / _CNT - mu * mu
    inv = jax.lax.rsqrt(var + 1e-5)
    o_ref[0, 0, 0] = ((t_ref[0, 0, 0] - mu) * inv * n1w_ref[...]
                      + n1b_ref[...] + x_ref[0, 0, 0])


def kernel(x, proj_w, proj_b, merge_w, merge_b, alpha, beta, norm0_w, norm0_b,
           lin0_w, lin0_b, lin1_w, lin1_b, norm1_w, norm1_b):
    f32 = jnp.float32
    x_q = (x.reshape(_N, _C, _FS, _HQ, _FS, _HQ)
           .transpose(0, 2, 4, 1, 3, 5).reshape(_N, _FS, _FS, _C, _HW))
    pb2 = proj_b.reshape(2 * _HID, 1)
    mw3 = merge_w.reshape(_C, _FC, _HALF).transpose(1, 0, 2)
    li = jnp.arange(_HW, dtype=jnp.int32)[:, None]
    ji = jnp.arange(_S, dtype=jnp.int32)[None, :]
    hi = li // _HQ
    wi = li - hi * _HQ
    q_pool = jnp.where((hi // _KH) * _CS + (wi // _KH) == ji,
                       1.0 / (_KH * _KH), 0.0).astype(jnp.bfloat16)
    mb2 = merge_b.reshape(_C, 1)
    a2 = alpha.reshape(1, 1)
    b2 = beta.reshape(1, 1)

    smem = pl.BlockSpec(memory_space=pltpu.SMEM)
    merged, s1, s2 = pl.pallas_call(
        _cluster_body,
        grid=(_N, _FS, _FS, _FC),
        in_specs=[
            smem, smem,
            pl.BlockSpec((1, 1, 1, _C, _HW), lambda n, h, w, f: (n, h, w, 0, 0)),
            pl.BlockSpec((_CG, _C), lambda n, h, w, f: (f, 0)),
            pl.BlockSpec((_CG, 1), lambda n, h, w, f: (f, 0)),
            pl.BlockSpec((1, _C, _HALF), lambda n, h, w, f: (f, 0, 0)),
            pl.BlockSpec((_C, 1), lambda n, h, w, f: (0, 0)),
            pl.BlockSpec((_HW, _S), lambda n, h, w, f: (0, 0)),
        ],
        out_specs=[
            pl.BlockSpec((1, 1, 1, _C, _HW), lambda n, h, w, f: (n, h, w, 0, 0)),
            pl.BlockSpec((1, 1, 1), lambda n, h, w, f: (_flat3(n, h, w), 0, 0),
                         memory_space=pltpu.SMEM),
            pl.BlockSpec((1, 1, 1), lambda n, h, w, f: (_flat3(n, h, w), 0, 0),
                         memory_space=pltpu.SMEM),
        ],
        out_shape=[
            jax.ShapeDtypeStruct((_N, _FS, _FS, _C, _HW), f32),
            jax.ShapeDtypeStruct((_N * _FS * _FS, 1, 1), f32),
            jax.ShapeDtypeStruct((_N * _FS * _FS, 1, 1), f32),
        ],
        scratch_shapes=[pltpu.VMEM((_C, _HW), jnp.bfloat16)],
        compiler_params=pltpu.CompilerParams(
            dimension_semantics=("parallel", "parallel", "parallel", "arbitrary")),
    )(a2, b2, x_q, proj_w, pb2, mw3, mb2, q_pool)

    t_all, p1, p2 = pl.pallas_call(
        _mlp_body,
        grid=(_N, _FS, _FS, _KT),
        in_specs=[
            smem, smem,
            pl.BlockSpec((1, 1, 1, _C, _TW), lambda n, h, w, k: (n, h, w, 0, k)),
            pl.BlockSpec((1, 1, 1, _C, _TW), lambda n, h, w, k: (n, h, w, 0, k)),
            pl.BlockSpec((_C, 1), lambda n, h, w, k: (0, 0)),
            pl.BlockSpec((_C, 1), lambda n, h, w, k: (0, 0)),
            pl.BlockSpec((2 * _C, 2 * _C), lambda n, h, w, k: (0, 0)),
            pl.BlockSpec((2 * _C, 1), lambda n, h, w, k: (0, 0)),
            pl.BlockSpec((_C, 2 * _C), lambda n, h, w, k: (0, 0)),
            pl.BlockSpec((_C, 1), lambda n, h, w, k: (0, 0)),
        ],
        out_specs=[
            pl.BlockSpec((1, 1, 1, _C, _TW), lambda n, h, w, k: (n, h, w, 0, k)),
            pl.BlockSpec((1, 1, 1), lambda n, h, w, k: (_flat3(n, h, w, k), 0, 0),
                         memory_space=pltpu.SMEM),
            pl.BlockSpec((1, 1, 1), lambda n, h, w, k: (_flat3(n, h, w, k), 0, 0),
                         memory_space=pltpu.SMEM),
        ],
        out_shape=[
            jax.ShapeDtypeStruct((_N, _FS, _FS, _C, _HW), f32),
            jax.ShapeDtypeStruct((_N * _FS * _FS * _KT, 1, 1), f32),
            jax.ShapeDtypeStruct((_N * _FS * _FS * _KT, 1, 1), f32),
        ],
        compiler_params=pltpu.CompilerParams(
            dimension_semantics=("parallel", "parallel", "parallel", "parallel")),
    )(s1, s2, x_q, merged, norm0_w.reshape(_C, 1), norm0_b.reshape(_C, 1),
      lin0_w, lin0_b.reshape(2 * _C, 1), lin1_w, lin1_b.reshape(_C, 1))

    out_q = pl.pallas_call(
        _final_body,
        grid=(_N, _FS, _FS, _KT),
        in_specs=[
            smem, smem,
            pl.BlockSpec((_C, 1), lambda n, h, w, k: (0, 0)),
            pl.BlockSpec((_C, 1), lambda n, h, w, k: (0, 0)),
            pl.BlockSpec((1, 1, 1, _C, _TW), lambda n, h, w, k: (n, h, w, 0, k)),
            pl.BlockSpec((1, 1, 1, _C, _TW), lambda n, h, w, k: (n, h, w, 0, k)),
        ],
        out_specs=pl.BlockSpec((1, 1, 1, _C, _TW),
                               lambda n, h, w, k: (n, h, w, 0, k)),
        out_shape=jax.ShapeDtypeStruct((_N, _FS, _FS, _C, _HW), f32),
        compiler_params=pltpu.CompilerParams(
            dimension_semantics=("parallel", "parallel", "parallel", "parallel")),
    )(p1, p2, norm1_w.reshape(_C, 1), norm1_b.reshape(_C, 1), t_all, x_q)

    return (out_q.reshape(_N, _FS, _FS, _C, _HQ, _HQ)
            .transpose(0, 3, 1, 4, 2, 5).reshape(_N, _C, _H, _W))


# speedup vs baseline: 5.6235x; 1.2409x over previous
import jax
import jax.numpy as jnp
from jax.experimental import pallas as pl
from jax.experimental.pallas import tpu as pltpu

_N, _C, _H, _W = 2, 384, 112, 112
_HID = 384
_FC = 8
_CS = 7
_FS = 2
_HQ = _H // _FS
_HW = _HQ * _HQ
_CG = 2 * _HID // _FC
_HALF = _CG // 2
_S = _CS * _CS
_KH = _HQ // _CS
_CNT = _C * _H * _W


def _dot3(a, b, dims):
    f32 = jnp.float32
    bf = jnp.bfloat16
    ah = a.astype(bf)
    al = (a - ah.astype(f32)).astype(bf)
    bh = b.astype(bf)
    bl = (b - bh.astype(f32)).astype(bf)
    return (jax.lax.dot_general(ah, bh, dims, preferred_element_type=f32)
            + jax.lax.dot_general(ah, bl, dims, preferred_element_type=f32)
            + jax.lax.dot_general(al, bh, dims, preferred_element_type=f32))


def _cluster_body(alpha_ref, beta_ref, x_ref, pw_ref, pb_ref, mw_ref, mb_ref,
                  q_ref, merged_ref, s1_ref, s2_ref, nxa_ref):
    bf = jnp.bfloat16
    f32 = jnp.float32
    xqb = x_ref[0, 0, 0].astype(bf)
    q = q_ref[...]
    alpha = alpha_ref[0, 0]
    beta = beta_ref[0, 0]
    ri = jax.lax.broadcasted_iota(jnp.int32, (_S, _S), 0)
    ci = jax.lax.broadcasted_iota(jnp.int32, (_S, _S), 1)
    ltri = jnp.where(ri >= ci, 1.0, 0.0).astype(bf)

    for fci in range(_FC):
        wg = pw_ref[fci * _CG:(fci + 1) * _CG, :]
        bg = pb_ref[fci * _CG:(fci + 1) * _CG, :]
        y = jnp.dot(wg.astype(bf), xqb,
                    preferred_element_type=f32) + bg

        yh = y.astype(bf)
        r1 = y - yh.astype(f32)
        ym = r1.astype(bf)
        yl = (r1 - ym.astype(f32)).astype(bf)
        cen = (jnp.dot(yh, q, preferred_element_type=f32)
               + jnp.dot(ym, q, preferred_element_type=f32)
               + jnp.dot(yl, q, preferred_element_type=f32))

        xp = y[:_HALF, :]
        xv = y[_HALF:, :]
        cp = cen[:_HALF, :]
        cv = cen[_HALF:, :]

        dx = jnp.maximum(jnp.sqrt(jnp.sum(xp * xp, axis=0, keepdims=True)),
                         1e-12)
        nx = xp / dx
        dc = jnp.maximum(jnp.sqrt(jnp.sum(cp * cp, axis=0, keepdims=True)),
                         1e-12)
        nc = cp / dc

        simT = jax.lax.dot_general(nc.astype(bf), nx.astype(bf),
                                   (((0,), (0,)), ((), ())),
                                   preferred_element_type=f32)
        simT = jax.nn.sigmoid(alpha * simT + beta)
        vals = jnp.max(simT, axis=0, keepdims=True)
        ismax = (simT >= vals).astype(f32)
        cum = jnp.dot(ltri, ismax.astype(bf), preferred_element_type=f32)
        w1 = jnp.where(cum == 1.0, ismax, 0.0) * vals

        xve = jnp.concatenate([xv, jnp.ones((1, _HW), f32)], axis=0)
        delta = _dot3(xve, w1, (((1,), (1,)), ((), ())))
        ncf = (cv + delta[:_HALF, :]) / (1.0 + delta[_HALF:, :])
        nxq = _dot3(ncf, w1, (((1,), (0,)), ((), ())))
        nxa_ref[fci * _HALF:(fci + 1) * _HALF, :] = nxq.astype(bf)

    merged = (jnp.dot(mw_ref[...].astype(bf), nxa_ref[...],
                      preferred_element_type=f32) + mb_ref[...])
    merged_ref[0, 0, 0] = merged.astype(bf)
    s1_ref[0, 0, 0] = jnp.sum(merged)
    s2_ref[0, 0, 0] = jnp.sum(merged * merged)


def _flat3(n, h, w):
    return n * 4 + h * 2 + w


def _mlp_body(s1_ref, s2_ref, x_ref, m_ref, n0w_ref, n0b_ref, w0_ref, b0_ref,
              w1_ref, b1_ref, t_ref, p1_ref, p2_ref):
    ni = pl.program_id(0)
    bf = jnp.bfloat16
    f32 = jnp.float32
    tot = 0.0
    totsq = 0.0
    for i in range(2):
        for j in range(2):
            tot = tot + s1_ref[ni * 4 + i * 2 + j, 0, 0]
            totsq = totsq + s2_ref[ni * 4 + i * 2 + j, 0, 0]
    mu = tot / _CNT
    var = totsq / _CNT - mu * mu
    inv = jax.lax.rsqrt(var + 1e-5)

    g = ((m_ref[0, 0, 0].astype(f32) - mu) * inv * n0w_ref[...]
         + n0b_ref[...])
    xt = x_ref[0, 0, 0]
    w0 = w0_ref[...].astype(bf)
    h1 = (jnp.dot(w0[:, :_C], xt.astype(bf), preferred_element_type=f32)
          + jnp.dot(w0[:, _C:], g.astype(bf), preferred_element_type=f32)
          + b0_ref[...])
    h1 = 0.5 * h1 * (1.0 + jax.lax.erf(h1 * 0.7071067811865476))
    t = (jnp.dot(w1_ref[...].astype(bf), h1.astype(bf),
                 preferred_element_type=f32) + b1_ref[...])
    t_ref[0, 0, 0] = t.astype(bf)
    p1_ref[0, 0, 0] = jnp.sum(t)
    p2_ref[0, 0, 0] = jnp.sum(t * t)


def _final_body(p1_ref, p2_ref, n1w_ref, n1b_ref, t_ref, x_ref, o_ref):
    ni = pl.program_id(0)
    tot = 0.0
    totsq = 0.0
    for i in range(2):
        for j in range(2):
            tot = tot + p1_ref[ni * 4 + i * 2 + j, 0, 0]
            totsq = totsq + p2_ref[ni * 4 + i * 2 + j, 0, 0]
    mu = tot / _CNT
    var = totsq / _CNT - mu * mu
    inv = jax.lax.rsqrt(var + 1e-5)
    o_ref[0, 0, 0] = ((t_ref[0, 0, 0].astype(jnp.float32) - mu) * inv
                      * n1w_ref[...] + n1b_ref[...] + x_ref[0, 0, 0])


def kernel(x, proj_w, proj_b, merge_w, merge_b, alpha, beta, norm0_w, norm0_b,
           lin0_w, lin0_b, lin1_w, lin1_b, norm1_w, norm1_b):
    f32 = jnp.float32
    bf = jnp.bfloat16
    x_q = (x.reshape(_N, _C, _FS, _HQ, _FS, _HQ)
           .transpose(0, 2, 4, 1, 3, 5).reshape(_N, _FS, _FS, _C, _HW))
    pb2 = proj_b.reshape(2 * _HID, 1)
    mb2 = merge_b.reshape(_C, 1)
    a2 = alpha.reshape(1, 1)
    b2 = beta.reshape(1, 1)
    li = jnp.arange(_HW, dtype=jnp.int32)[:, None]
    ji = jnp.arange(_S, dtype=jnp.int32)[None, :]
    hi = li // _HQ
    wi = li - hi * _HQ
    q_pool = jnp.where((hi // _KH) * _CS + (wi // _KH) == ji,
                       1.0 / (_KH * _KH), 0.0).astype(bf)

    smem = pl.BlockSpec(memory_space=pltpu.SMEM)
    merged, s1, s2 = pl.pallas_call(
        _cluster_body,
        grid=(_N, _FS, _FS),
        in_specs=[
            smem, smem,
            pl.BlockSpec((1, 1, 1, _C, _HW), lambda n, h, w: (n, h, w, 0, 0)),
            pl.BlockSpec((2 * _HID, _C), lambda n, h, w: (0, 0)),
            pl.BlockSpec((2 * _HID, 1), lambda n, h, w: (0, 0)),
            pl.BlockSpec((_C, _C), lambda n, h, w: (0, 0)),
            pl.BlockSpec((_C, 1), lambda n, h, w: (0, 0)),
            pl.BlockSpec((_HW, _S), lambda n, h, w: (0, 0)),
        ],
        out_specs=[
            pl.BlockSpec((1, 1, 1, _C, _HW), lambda n, h, w: (n, h, w, 0, 0)),
            pl.BlockSpec((1, 1, 1), lambda n, h, w: (_flat3(n, h, w), 0, 0),
                         memory_space=pltpu.SMEM),
            pl.BlockSpec((1, 1, 1), lambda n, h, w: (_flat3(n, h, w), 0, 0),
                         memory_space=pltpu.SMEM),
        ],
        out_shape=[
            jax.ShapeDtypeStruct((_N, _FS, _FS, _C, _HW), bf),
            jax.ShapeDtypeStruct((_N * _FS * _FS, 1, 1), f32),
            jax.ShapeDtypeStruct((_N * _FS * _FS, 1, 1), f32),
        ],
        scratch_shapes=[pltpu.VMEM((_C, _HW), bf)],
        compiler_params=pltpu.CompilerParams(
            dimension_semantics=("parallel", "parallel", "parallel")),
    )(a2, b2, x_q, proj_w, pb2, merge_w, mb2, q_pool)

    t_all, p1, p2 = pl.pallas_call(
        _mlp_body,
        grid=(_N, _FS, _FS),
        in_specs=[
            smem, smem,
            pl.BlockSpec((1, 1, 1, _C, _HW), lambda n, h, w: (n, h, w, 0, 0)),
            pl.BlockSpec((1, 1, 1, _C, _HW), lambda n, h, w: (n, h, w, 0, 0)),
            pl.BlockSpec((_C, 1), lambda n, h, w: (0, 0)),
            pl.BlockSpec((_C, 1), lambda n, h, w: (0, 0)),
            pl.BlockSpec((2 * _C, 2 * _C), lambda n, h, w: (0, 0)),
            pl.BlockSpec((2 * _C, 1), lambda n, h, w: (0, 0)),
            pl.BlockSpec((_C, 2 * _C), lambda n, h, w: (0, 0)),
            pl.BlockSpec((_C, 1), lambda n, h, w: (0, 0)),
        ],
        out_specs=[
            pl.BlockSpec((1, 1, 1, _C, _HW), lambda n, h, w: (n, h, w, 0, 0)),
            pl.BlockSpec((1, 1, 1), lambda n, h, w: (_flat3(n, h, w), 0, 0),
                         memory_space=pltpu.SMEM),
            pl.BlockSpec((1, 1, 1), lambda n, h, w: (_flat3(n, h, w), 0, 0),
                         memory_space=pltpu.SMEM),
        ],
        out_shape=[
            jax.ShapeDtypeStruct((_N, _FS, _FS, _C, _HW), bf),
            jax.ShapeDtypeStruct((_N * _FS * _FS, 1, 1), f32),
            jax.ShapeDtypeStruct((_N * _FS * _FS, 1, 1), f32),
        ],
        compiler_params=pltpu.CompilerParams(
            dimension_semantics=("parallel", "parallel", "parallel")),
    )(s1, s2, x_q, merged, norm0_w.reshape(_C, 1), norm0_b.reshape(_C, 1),
      lin0_w, lin0_b.reshape(2 * _C, 1), lin1_w, lin1_b.reshape(_C, 1))

    out_q = pl.pallas_call(
        _final_body,
        grid=(_N, _FS, _FS),
        in_specs=[
            smem, smem,
            pl.BlockSpec((_C, 1), lambda n, h, w: (0, 0)),
            pl.BlockSpec((_C, 1), lambda n, h, w: (0, 0)),
            pl.BlockSpec((1, 1, 1, _C, _HW), lambda n, h, w: (n, h, w, 0, 0)),
            pl.BlockSpec((1, 1, 1, _C, _HW), lambda n, h, w: (n, h, w, 0, 0)),
        ],
        out_specs=pl.BlockSpec((1, 1, 1, _C, _HW),
                               lambda n, h, w: (n, h, w, 0, 0)),
        out_shape=jax.ShapeDtypeStruct((_N, _FS, _FS, _C, _HW), f32),
        compiler_params=pltpu.CompilerParams(
            dimension_semantics=("parallel", "parallel", "parallel")),
    )(p1, p2, norm1_w.reshape(_C, 1), norm1_b.reshape(_C, 1), t_all, x_q)

    return (out_q.reshape(_N, _FS, _FS, _C, _HQ, _HQ)
            .transpose(0, 3, 1, 4, 2, 5).reshape(_N, _C, _H, _W))
